# baseline (speedup 1.0000x reference)
import jax
import jax.numpy as jnp
from jax.experimental import pallas as pl
from jax.experimental.pallas import tpu as pltpu

F = 128
BLK = 400


def _gcn_body(adj_ref, x_ref, W1_ref, b1_ref, W2_ref, b2_ref, out_ref,
              s0_ref, s1_ref):
    l = pl.program_id(0)
    i = pl.program_id(1)

    @pl.when((l == 0) & (i == 0))
    def _init():
        s0_ref[...] = jnp.dot(x_ref[...].astype(jnp.bfloat16),
                              W1_ref[...].astype(jnp.bfloat16),
                              preferred_element_type=jnp.float32
                              ).astype(jnp.bfloat16)

    @pl.when(l == 0)
    def _layer0():
        adj = adj_ref[0].astype(jnp.bfloat16)
        h = jnp.dot(adj, s0_ref[...], preferred_element_type=jnp.float32)
        h = jnp.maximum(h + b1_ref[...], 0.0).astype(jnp.bfloat16)
        s1 = jnp.dot(h, W2_ref[...].astype(jnp.bfloat16),
                     preferred_element_type=jnp.float32)
        s1_ref[pl.ds(i * BLK, BLK), :] = s1.astype(jnp.bfloat16)

    @pl.when(l == 1)
    def _layer1():
        adj = adj_ref[0].astype(jnp.bfloat16)
        out_ref[...] = jnp.dot(adj, s1_ref[...],
                               preferred_element_type=jnp.float32) + b2_ref[...]


def kernel(x, adjs, W1, b1, W2, b2):
    n = x.shape[0]
    nb = n // BLK
    b1r = b1.reshape(1, F)
    b2r = b2.reshape(1, F)
    return pl.pallas_call(
        _gcn_body,
        grid=(2, nb),
        in_specs=[
            pl.BlockSpec((1, BLK, n), lambda l, i: (l, i, 0)),
            pl.BlockSpec((n, F), lambda l, i: (0, 0)),
            pl.BlockSpec((F, F), lambda l, i: (0, 0)),
            pl.BlockSpec((1, F), lambda l, i: (0, 0)),
            pl.BlockSpec((F, F), lambda l, i: (0, 0)),
            pl.BlockSpec((1, F), lambda l, i: (0, 0)),
        ],
        out_specs=pl.BlockSpec((BLK, F), lambda l, i: (i * l, 0)),
        out_shape=jax.ShapeDtypeStruct((n, F), jnp.float32),
        scratch_shapes=[
            pltpu.VMEM((n, F), jnp.bfloat16),
            pltpu.VMEM((n, F), jnp.bfloat16),
        ],
    )(adjs, x, W1, b1r, W2, b2r)
